# Initial kernel scaffold; baseline (speedup 1.0000x reference)
import jax
import jax.numpy as jnp
from jax.experimental import pallas as pl


def kernel(x, edge_index, edge_attr, weights_matrices, bias, inputSize, outputSize):
    raise NotImplementedError("write your pallas kernel here")


# speedup vs baseline: 1.2239x; 1.2239x over previous
import functools

import jax
import jax.numpy as jnp
from jax import lax
from jax.experimental import pallas as pl
from jax.experimental.pallas import tpu as pltpu
from jax.experimental.pallas import tpu_sc as plsc

NC = 2
NS = 16
NW = NC * NS
L = 16
C = 128


@functools.lru_cache(maxsize=None)
def _sc_fn(N, E, IN_C, OUT_C):
    assert IN_C == L and OUT_C == L
    W2 = OUT_C * IN_C
    n_chunks = E // C
    assert n_chunks * C == E
    base_trips = n_chunks // NW
    extra = n_chunks - base_trips * NW
    RP = (N // NS) & ~7
    rem_rows = N - RP * NS
    assert rem_rows % 8 == 0
    rem_tiles = rem_rows // 8

    mesh = plsc.VectorSubcoreMesh(core_axis_name="c", subcore_axis_name="s")

    @functools.partial(
        pl.kernel,
        out_type=jax.ShapeDtypeStruct((NC * N, OUT_C), jnp.float32),
        mesh=mesh,
        scratch_types=[
            pltpu.VMEM((C,), jnp.int32),
            pltpu.VMEM((C,), jnp.int32),
            pltpu.VMEM((C, IN_C), jnp.float32),
            pltpu.VMEM((C * W2,), jnp.float32),
            pltpu.VMEM((C, OUT_C), jnp.float32),
            pltpu.VMEM((RP, OUT_C), jnp.float32),
            pltpu.VMEM_SHARED((N, OUT_C), jnp.float32),
            pltpu.SemaphoreType.DMA,
        ],
        compiler_params=pltpu.CompilerParams(
            needs_layout_passes=False, use_tc_tiling_on_sc=False),
    )
    def body(x_hbm, src_hbm, dst_hbm, w_hbm, part_hbm,
             idx_s, idx_d, xj, wbuf, msg, zbuf, acc, sem):
        cid = lax.axis_index("c")
        sid = lax.axis_index("s")
        wid = sid * NC + cid

        def zrow(j, carry):
            zbuf[j, :] = jnp.zeros((OUT_C,), jnp.float32)
            return carry

        lax.fori_loop(0, RP, zrow, 0)
        pltpu.sync_copy(zbuf, acc.at[pl.ds(sid * RP, RP)])

        @pl.when(sid < rem_tiles)
        def _():
            pltpu.sync_copy(zbuf.at[pl.ds(0, 8)],
                            acc.at[pl.ds(NS * RP + sid * 8, 8)])

        plsc.subcore_barrier()

        col_iota = lax.iota(jnp.int32, L) * IN_C

        def chunk(c, carry):
            base = (c * NW + wid) * C
            pltpu.sync_copy(src_hbm.at[pl.ds(base, C)], idx_s)
            pltpu.sync_copy(dst_hbm.at[pl.ds(base, C)], idx_d)
            pltpu.async_copy(x_hbm.at[idx_s], xj, sem).wait()
            pltpu.sync_copy(w_hbm.at[pl.ds(base * W2, C * W2)], wbuf)

            def edge(e, carry2):
                bw = e * W2
                xrow = xj[e, :]
                accv = jnp.zeros((L,), jnp.float32)
                for i in range(IN_C):
                    col = plsc.load_gather(wbuf, [col_iota + (bw + i)])
                    accv = accv + col * xrow[i]
                msg[e, :] = accv
                return carry2

            lax.fori_loop(0, C, edge, 0)
            pltpu.sync_copy(msg, acc.at[idx_d], add=True)
            return carry

        trips = jnp.where(wid < extra, base_trips + 1, base_trips)
        lax.fori_loop(0, trips, chunk, 0)

        plsc.subcore_barrier()
        pltpu.sync_copy(acc.at[pl.ds(sid * RP, RP)],
                        part_hbm.at[pl.ds(cid * N + sid * RP, RP)])

        @pl.when(sid < rem_tiles)
        def _():
            pltpu.sync_copy(acc.at[pl.ds(NS * RP + sid * 8, 8)],
                            part_hbm.at[pl.ds(cid * N + NS * RP + sid * 8, 8)])

    return body


def _combine(p_ref, b_ref, o_ref):
    o_ref[...] = jnp.maximum(p_ref[0] + p_ref[1] + b_ref[...], 0.0)


@functools.lru_cache(maxsize=None)
def _combine_fn(rows):
    return pl.pallas_call(
        _combine,
        out_shape=jax.ShapeDtypeStruct((rows, 128), jnp.float32),
    )


def kernel(x, edge_index, edge_attr, weights_matrices, bias, inputSize, outputSize):
    N, in_c = x.shape
    E, out_c, _ = weights_matrices.shape
    src = edge_index[0]
    dst = edge_index[1]
    w_flat = weights_matrices.reshape(E * out_c * in_c)
    partials = _sc_fn(N, E, in_c, out_c)(x, src, dst, w_flat)
    partials = partials.reshape(NC, N, out_c)
    per_row = 128 // out_c
    rows = N // per_row
    p = partials.reshape(NC, rows, 128)
    bias_t = jnp.tile(bias, per_row).reshape(1, 128)
    out = _combine_fn(rows)(p, bias_t)
    return out.reshape(N, out_c)
